# Initial kernel scaffold; baseline (speedup 1.0000x reference)
import jax
import jax.numpy as jnp
from jax.experimental import pallas as pl


def kernel(feat, edge_index, weight, bias):
    raise NotImplementedError("write your pallas kernel here")


# speedup vs baseline: 5.1913x; 5.1913x over previous
import functools

import jax
import jax.numpy as jnp
from jax import lax
from jax.experimental import pallas as pl
from jax.experimental.pallas import tpu as pltpu
from jax.experimental.pallas import tpu_sc as plsc

NC = 2
NS = 16
NW = NC * NS
CHUNK = 128


def _sc_mesh():
    return plsc.VectorSubcoreMesh(core_axis_name="c", subcore_axis_name="s")


def _degree_kernel(n_acc, c_chunks):
    rt = n_acc // NS

    def body(dst_hbm, zeros_hbm, ones_hbm, out_hbm, idx_v, ones_v, stage_v,
             deg_sp):
        cid = lax.axis_index("c")
        sid = lax.axis_index("s")
        w = cid * NS + sid
        r0 = sid * rt
        pltpu.sync_copy(zeros_hbm, stage_v)
        pltpu.sync_copy(stage_v, deg_sp.at[pl.ds(r0, rt)])
        pltpu.sync_copy(ones_hbm, ones_v)
        pltpu.sync_copy(dst_hbm.at[w], idx_v)
        plsc.subcore_barrier()

        def step(j, carry):
            pltpu.sync_copy(ones_v, deg_sp.at[idx_v.at[j]], add=True)
            return carry

        lax.fori_loop(0, c_chunks, step, 0)
        plsc.subcore_barrier()
        pltpu.sync_copy(deg_sp.at[pl.ds(r0, rt)], stage_v)
        pltpu.sync_copy(stage_v, out_hbm.at[pl.ds(cid * n_acc + r0, rt)])

    return pl.kernel(
        body,
        out_type=jax.ShapeDtypeStruct((NC * n_acc,), jnp.float32),
        mesh=_sc_mesh(),
        scratch_types=[
            pltpu.VMEM((c_chunks, CHUNK), jnp.int32),
            pltpu.VMEM((CHUNK,), jnp.float32),
            pltpu.VMEM((rt,), jnp.float32),
            pltpu.VMEM_SHARED((n_acc,), jnp.float32),
        ],
    )


def _aggregate_kernel(n_acc, c_chunks, d):
    rt = n_acc // NS

    row_chunks = []
    off = 0
    while off < rt:
        row_chunks.append((off, min(CHUNK, rt - off)))
        off += CHUNK

    def body(g_hbm, src_hbm, dst_hbm, zeros_hbm, out_hbm,
             src_v, dst_v, buf_v, gsem, acc_sp):
        cid = lax.axis_index("c")
        sid = lax.axis_index("s")
        w = cid * NS + sid
        r0 = sid * rt
        pltpu.sync_copy(zeros_hbm, buf_v)
        for off, cnt in row_chunks:
            pltpu.sync_copy(buf_v.at[pl.ds(0, cnt)],
                            acc_sp.at[pl.ds(r0 + off, cnt)])
        pltpu.sync_copy(src_hbm.at[w], src_v)
        pltpu.sync_copy(dst_hbm.at[w], dst_v)
        plsc.subcore_barrier()

        def step(j, carry):
            pltpu.async_copy(g_hbm.at[src_v.at[j]], buf_v, gsem).wait()
            pltpu.sync_copy(buf_v, acc_sp.at[dst_v.at[j]], add=True)
            return carry

        lax.fori_loop(0, c_chunks, step, 0)
        plsc.subcore_barrier()
        for off, cnt in row_chunks:
            pltpu.sync_copy(acc_sp.at[pl.ds(r0 + off, cnt)],
                            buf_v.at[pl.ds(0, cnt)])
            pltpu.sync_copy(buf_v.at[pl.ds(0, cnt)],
                            out_hbm.at[cid, pl.ds(r0 + off, cnt)])

    return pl.kernel(
        body,
        out_type=jax.ShapeDtypeStruct((NC, n_acc, d), jnp.float32),
        mesh=_sc_mesh(),
        scratch_types=[
            pltpu.VMEM((c_chunks, CHUNK), jnp.int32),
            pltpu.VMEM((c_chunks, CHUNK), jnp.int32),
            pltpu.VMEM((CHUNK, d), jnp.float32),
            pltpu.SemaphoreType.DMA,
            pltpu.VMEM_SHARED((n_acc, d), jnp.float32),
        ],
    )


def _row_block(n):
    for rb in (1000, 500, 250, 125, 100, 50, 25, 10, 5, 2):
        if n % rb == 0:
            return rb
    return n


def _scale_matmul(feat, d0, d1, weight):
    n, d_in = feat.shape
    d_out = weight.shape[1]
    rb = _row_block(n)

    def body(f_ref, d0_ref, d1_ref, w_ref, o_ref):
        scale = lax.rsqrt(d0_ref[...] + d1_ref[...])
        o_ref[...] = jnp.dot(f_ref[...] * scale, w_ref[...],
                             preferred_element_type=jnp.float32,
                             precision=lax.Precision.HIGHEST)

    return pl.pallas_call(
        body,
        grid=(n // rb,),
        in_specs=[
            pl.BlockSpec((rb, d_in), lambda i: (i, 0)),
            pl.BlockSpec((rb, 1), lambda i: (i, 0)),
            pl.BlockSpec((rb, 1), lambda i: (i, 0)),
            pl.BlockSpec((d_in, d_out), lambda i: (0, 0)),
        ],
        out_specs=pl.BlockSpec((rb, d_out), lambda i: (i, 0)),
        out_shape=jax.ShapeDtypeStruct((n, d_out), jnp.float32),
    )(feat, d0, d1, weight)


def _finalize(p0, p1, d0, d1, bias2):
    n, d = p0.shape
    rb = _row_block(n)

    def body(p0_ref, p1_ref, d0_ref, d1_ref, b_ref, o_ref):
        scale = lax.rsqrt(d0_ref[...] + d1_ref[...])
        o_ref[...] = (p0_ref[...] + p1_ref[...]) * scale + b_ref[...]

    return pl.pallas_call(
        body,
        grid=(n // rb,),
        in_specs=[
            pl.BlockSpec((rb, d), lambda i: (i, 0)),
            pl.BlockSpec((rb, d), lambda i: (i, 0)),
            pl.BlockSpec((rb, 1), lambda i: (i, 0)),
            pl.BlockSpec((rb, 1), lambda i: (i, 0)),
            pl.BlockSpec((1, d), lambda i: (0, 0)),
        ],
        out_specs=pl.BlockSpec((rb, d), lambda i: (i, 0)),
        out_shape=jax.ShapeDtypeStruct((n, d), jnp.float32),
    )(p0, p1, d0, d1, bias2)


def kernel(feat, edge_index, weight, bias):
    n, d_in = feat.shape
    d_out = weight.shape[1]
    e = edge_index.shape[1]

    src = edge_index[0].astype(jnp.int32)
    dst = edge_index[1].astype(jnp.int32)

    per_w = -(-e // (NW * CHUNK)) * CHUNK
    c_chunks = per_w // CHUNK
    e_pad = per_w * NW
    pad = e_pad - e
    src3 = jnp.concatenate([src, jnp.zeros((pad,), jnp.int32)]).reshape(
        NW, c_chunks, CHUNK)
    dst3 = jnp.concatenate([dst, jnp.full((pad,), n, jnp.int32)]).reshape(
        NW, c_chunks, CHUNK)

    n_acc = -(-(n + 1) // (NS * 8)) * (NS * 8)
    rt = n_acc // NS

    zeros1 = jnp.zeros((rt,), jnp.float32)
    ones1 = jnp.ones((CHUNK,), jnp.float32)
    zeros2 = jnp.zeros((CHUNK, d_out), jnp.float32)

    degp = _degree_kernel(n_acc, c_chunks)(dst3, zeros1, ones1)
    d0 = degp[:n].reshape(n, 1)
    d1 = degp[n_acc:n_acc + n].reshape(n, 1)

    g = _scale_matmul(feat, d0, d1, weight)

    partials = _aggregate_kernel(n_acc, c_chunks, d_out)(g, src3, dst3, zeros2)

    return _finalize(partials[0, :n], partials[1, :n], d0, d1,
                     bias.reshape(1, d_out))
